# baseline (speedup 1.0000x reference)
import functools

import jax
import jax.numpy as jnp
from jax import lax
from jax.experimental import pallas as pl
from jax.experimental.pallas import tpu as pltpu
from jax.experimental.pallas import tpu_sc as plsc

NUM_CLASSES = 1000
N_CTX = 16
CTX_DIM = 512
SEQ_LEN = 77
SUFFIX_LEN = SEQ_LEN - 1 - N_CTX

CTX_D = N_CTX * CTX_DIM
SUF_D = SUFFIX_LEN * CTX_DIM
ROW_D = SEQ_LEN * CTX_DIM

NC = 2
NS = 16
NW = NC * NS

CH = 2


def _make_sc_kernel(B: int):
    b_per_w = B // NW
    n_chunks = b_per_w // CH
    mesh = plsc.VectorSubcoreMesh(
        core_axis_name="c", subcore_axis_name="s", num_cores=NC, num_subcores=NS
    )

    @functools.partial(
        pl.kernel,
        out_type=jax.ShapeDtypeStruct((B * ROW_D,), jnp.float32),
        mesh=mesh,
        scratch_types=[
            pltpu.VMEM((n_chunks, CH), jnp.int32),
            pltpu.VMEM((CTX_DIM,), jnp.float32),
            pltpu.VMEM((CH, CTX_D), jnp.float32),
            pltpu.VMEM((CH, SUF_D), jnp.float32),
            pltpu.SemaphoreType.DMA,
        ],
    )
    def body(label_hbm, ctx_hbm, prefix_hbm, suffix_hbm, out_hbm,
             idx_v, pre_v, ctx_v, suf_v, sem):
        wid = lax.axis_index("s") * NC + lax.axis_index("c")
        base = wid * b_per_w
        pltpu.sync_copy(label_hbm.at[wid], idx_v)
        pltpu.sync_copy(prefix_hbm, pre_v)

        def step(i, _):
            idx = idx_v.at[i]
            cg = pltpu.async_copy(ctx_hbm.at[idx], ctx_v, sem)
            sg = pltpu.async_copy(suffix_hbm.at[idx], suf_v, sem)
            cg.wait()
            sg.wait()
            row0 = (base + i * CH) * ROW_D
            for j in range(CH):
                off = row0 + j * ROW_D
                pltpu.sync_copy(pre_v, out_hbm.at[pl.ds(off, CTX_DIM)])
                pltpu.sync_copy(
                    ctx_v.at[j], out_hbm.at[pl.ds(off + CTX_DIM, CTX_D)]
                )
                pltpu.sync_copy(
                    suf_v.at[j],
                    out_hbm.at[pl.ds(off + CTX_DIM + CTX_D, SUF_D)],
                )
            return 0

        lax.fori_loop(0, n_chunks, step, 0)

    return body


def kernel(label, cls_ctx, token_prefix, token_suffix):
    B = label.shape[0]
    label3 = label.astype(jnp.int32).reshape(NW, B // (NW * CH), CH)
    ctx2 = cls_ctx.reshape(NUM_CLASSES, CTX_D)
    suf2 = token_suffix.reshape(NUM_CLASSES, SUF_D)
    prefix1 = token_prefix.reshape(CTX_DIM)
    out = _make_sc_kernel(B)(label3, ctx2, prefix1, suf2)
    return out.reshape(B, SEQ_LEN, CTX_DIM)


# speedup vs baseline: 1.0288x; 1.0288x over previous
import functools

import jax
import jax.numpy as jnp
from jax import lax
from jax.experimental import pallas as pl
from jax.experimental.pallas import tpu as pltpu
from jax.experimental.pallas import tpu_sc as plsc

NUM_CLASSES = 1000
N_CTX = 16
CTX_DIM = 512
SEQ_LEN = 77
SUFFIX_LEN = SEQ_LEN - 1 - N_CTX

CTX_D = N_CTX * CTX_DIM
SUF_D = SUFFIX_LEN * CTX_DIM
ROW_D = SEQ_LEN * CTX_DIM

NC = 2
NS = 16
NW = NC * NS


def _make_sc_kernel(B: int):
    b_per_w = B // NW
    mesh = plsc.VectorSubcoreMesh(
        core_axis_name="c", subcore_axis_name="s", num_cores=NC, num_subcores=NS
    )

    @functools.partial(
        pl.kernel,
        out_type=jax.ShapeDtypeStruct((B * ROW_D,), jnp.float32),
        mesh=mesh,
        scratch_types=[
            pltpu.VMEM((b_per_w, 1), jnp.int32),
            pltpu.VMEM((CTX_DIM,), jnp.float32),
            pltpu.VMEM((2, 1, CTX_D), jnp.float32),
            pltpu.VMEM((2, 1, SUF_D), jnp.float32),
            pltpu.SemaphoreType.DMA,
            pltpu.SemaphoreType.DMA,
            pltpu.SemaphoreType.DMA,
            pltpu.SemaphoreType.DMA,
        ],
    )
    def body(label_hbm, ctx_hbm, prefix_hbm, suffix_hbm, out_hbm,
             idx_v, pre_v, ctx_v, suf_v, gsem0, gsem1, wsem0, wsem1):
        gsem = (gsem0, gsem1)
        wsem = (wsem0, wsem1)
        wid = lax.axis_index("s") * NC + lax.axis_index("c")
        base = wid * b_per_w
        pltpu.sync_copy(label_hbm.at[wid], idx_v)
        pltpu.sync_copy(prefix_hbm, pre_v)

        def g_copies(i, k):
            idx = idx_v.at[i]
            return (
                pltpu.make_async_copy(ctx_hbm.at[idx], ctx_v.at[k], gsem[k]),
                pltpu.make_async_copy(suffix_hbm.at[idx], suf_v.at[k], gsem[k]),
            )

        def w_copies(i, k):
            off = (base + i) * ROW_D
            return (
                pltpu.make_async_copy(
                    pre_v, out_hbm.at[pl.ds(off, CTX_DIM)], wsem[k]),
                pltpu.make_async_copy(
                    ctx_v.at[k, 0],
                    out_hbm.at[pl.ds(off + CTX_DIM, CTX_D)], wsem[k]),
                pltpu.make_async_copy(
                    suf_v.at[k, 0],
                    out_hbm.at[pl.ds(off + CTX_DIM + CTX_D, SUF_D)], wsem[k]),
            )

        for k in (0, 1):
            for c in g_copies(k, k):
                c.start()

        def pair(g, _):
            for k in (0, 1):
                i = 2 * g + k
                for c in g_copies(i, k):
                    c.wait()
                wc = w_copies(i, k)
                for c in wc:
                    c.start()
                for c in wc:
                    c.wait()

                @pl.when(i + 2 < b_per_w)
                def _():
                    for c in g_copies(i + 2, k):
                        c.start()

            return 0

        lax.fori_loop(0, b_per_w // 2, pair, 0)

    return body


def kernel(label, cls_ctx, token_prefix, token_suffix):
    B = label.shape[0]
    label2 = label.astype(jnp.int32).reshape(NW, B // NW, 1)
    ctx2 = cls_ctx.reshape(NUM_CLASSES, CTX_D)
    suf2 = token_suffix.reshape(NUM_CLASSES, SUF_D)
    prefix1 = token_prefix.reshape(CTX_DIM)
    out = _make_sc_kernel(B)(label2, ctx2, prefix1, suf2)
    return out.reshape(B, SEQ_LEN, CTX_DIM)


# speedup vs baseline: 1.2560x; 1.2209x over previous
import functools

import jax
import jax.numpy as jnp
from jax import lax
from jax.experimental import pallas as pl
from jax.experimental.pallas import tpu as pltpu
from jax.experimental.pallas import tpu_sc as plsc

NUM_CLASSES = 1000
N_CTX = 16
CTX_DIM = 512
SEQ_LEN = 77
SUFFIX_LEN = SEQ_LEN - 1 - N_CTX
PAD_SEQ = 80

NC = 2
NS = 16
NW = NC * NS


def _make_sc_kernel(B: int):
    b_per_w = B // NW
    mesh = plsc.VectorSubcoreMesh(
        core_axis_name="c", subcore_axis_name="s", num_cores=NC, num_subcores=NS
    )

    @functools.partial(
        pl.kernel,
        out_type=jax.ShapeDtypeStruct((B, PAD_SEQ, CTX_DIM), jnp.float32),
        mesh=mesh,
        compiler_params=pltpu.CompilerParams(needs_layout_passes=False),
        scratch_types=[
            pltpu.VMEM((1, b_per_w), jnp.int32),
            pltpu.VMEM((16,), jnp.int32),
            pltpu.VMEM((16,), jnp.int32),
            pltpu.VMEM((1, PAD_SEQ, CTX_DIM), jnp.float32),
            pltpu.VMEM((1, PAD_SEQ, CTX_DIM), jnp.float32),
            pltpu.SemaphoreType.DMA,
            pltpu.SemaphoreType.DMA,
            pltpu.SemaphoreType.DMA,
            pltpu.SemaphoreType.DMA,
        ],
    )
    def body(label_hbm, table_hbm, out_hbm,
             idx_v, is0, is1, row0, row1, gsem0, gsem1, wsem0, wsem1):
        idx_s = (is0, is1)
        row_v = (row0, row1)
        gsem = (gsem0, gsem1)
        wsem = (wsem0, wsem1)
        wid = lax.axis_index("s") * NC + lax.axis_index("c")
        base = wid * b_per_w
        pltpu.sync_copy(label_hbm.at[wid], idx_v)
        zeros16 = jnp.zeros((16,), jnp.int32)

        def stage_idx(i, k):
            ivec = plsc.load_gather(
                idx_v, [zeros16, jnp.full((16,), i, jnp.int32)]
            )
            idx_s[k][...] = ivec

        def g_copy(k):
            return pltpu.make_async_copy(
                table_hbm.at[idx_s[k].at[pl.ds(0, 1)]], row_v[k], gsem[k]
            )

        def w_copy(i, k):
            return pltpu.make_async_copy(
                row_v[k], out_hbm.at[pl.ds(base + i, 1)], wsem[k]
            )

        for k in (0, 1):
            stage_idx(k, k)
            g_copy(k).start()

        def pair(g, _):
            for k in (0, 1):
                i = 2 * g + k
                g_copy(k).wait()
                wc = w_copy(i, k)
                wc.start()
                wc.wait()

                @pl.when(i + 2 < b_per_w)
                def _():
                    stage_idx(i + 2, k)
                    g_copy(k).start()

            return 0

        lax.fori_loop(0, b_per_w // 2, pair, 0)

    return body


def kernel(label, cls_ctx, token_prefix, token_suffix):
    B = label.shape[0]
    table = jnp.concatenate(
        [
            jnp.broadcast_to(token_prefix, (NUM_CLASSES, 1, CTX_DIM)),
            cls_ctx,
            token_suffix,
            jnp.zeros((NUM_CLASSES, PAD_SEQ - SEQ_LEN, CTX_DIM), jnp.float32),
        ],
        axis=1,
    )
    label3 = label.astype(jnp.int32).reshape(NW, 1, B // NW)
    out = _make_sc_kernel(B)(label3, table)
    return out[:, :SEQ_LEN, :]


# speedup vs baseline: 1.2567x; 1.0005x over previous
import functools

import jax
import jax.numpy as jnp
from jax import lax
from jax.experimental import pallas as pl
from jax.experimental.pallas import tpu as pltpu
from jax.experimental.pallas import tpu_sc as plsc

NUM_CLASSES = 1000
N_CTX = 16
CTX_DIM = 512
SEQ_LEN = 77
SUFFIX_LEN = SEQ_LEN - 1 - N_CTX
PAD_SEQ = 80

NC = 2
NS = 16
NW = NC * NS


def _make_sc_kernel(B: int):
    b_per_w = B // NW
    mesh = plsc.VectorSubcoreMesh(
        core_axis_name="c", subcore_axis_name="s", num_cores=NC, num_subcores=NS
    )

    @functools.partial(
        pl.kernel,
        out_type=jax.ShapeDtypeStruct((B, PAD_SEQ, CTX_DIM), jnp.float32),
        mesh=mesh,
        compiler_params=pltpu.CompilerParams(needs_layout_passes=False),
        scratch_types=[
            pltpu.VMEM((1, b_per_w), jnp.int32),
            pltpu.VMEM((16,), jnp.int32),
            pltpu.VMEM((16,), jnp.int32),
            pltpu.VMEM((1, PAD_SEQ, CTX_DIM), jnp.float32),
            pltpu.VMEM((1, PAD_SEQ, CTX_DIM), jnp.float32),
            pltpu.SemaphoreType.DMA,
            pltpu.SemaphoreType.DMA,
            pltpu.SemaphoreType.DMA,
            pltpu.SemaphoreType.DMA,
        ],
    )
    def body(label_hbm, table_hbm, out_hbm,
             idx_v, is0, is1, row0, row1, gsem0, gsem1, wsem0, wsem1):
        idx_s = (is0, is1)
        row_v = (row0, row1)
        gsem = (gsem0, gsem1)
        wsem = (wsem0, wsem1)
        wid = lax.axis_index("s") * NC + lax.axis_index("c")
        base = wid * b_per_w
        pltpu.sync_copy(label_hbm.at[wid], idx_v)
        zeros16 = jnp.zeros((16,), jnp.int32)

        def stage_idx(i, k):
            ivec = plsc.load_gather(
                idx_v, [zeros16, jnp.full((16,), i, jnp.int32)]
            )
            idx_s[k][...] = ivec

        def g_copy(k):
            return pltpu.make_async_copy(
                table_hbm.at[idx_s[k].at[pl.ds(0, 1)]], row_v[k], gsem[k]
            )

        def w_copy(i, k):
            return pltpu.make_async_copy(
                row_v[k], out_hbm.at[pl.ds(base + i, 1)], wsem[k]
            )

        for k in (0, 1):
            stage_idx(k, k)
            g_copy(k).start()

        def pair(g, _):
            for k in (0, 1):
                i = 2 * g + k
                g_copy(k).wait()
                wc = w_copy(i, k)
                wc.start()
                wc.wait()

                @pl.when(i + 2 < b_per_w)
                def _():
                    stage_idx(i + 2, k)
                    g_copy(k).start()

            return 0

        lax.fori_loop(0, b_per_w // 2, pair, 0)

    return body


def kernel(label, cls_ctx, token_prefix, token_suffix):
    B = label.shape[0]
    table = (
        jnp.pad(jnp.broadcast_to(token_prefix, (NUM_CLASSES, 1, CTX_DIM)),
                ((0, 0), (0, PAD_SEQ - 1), (0, 0)))
        + jnp.pad(cls_ctx, ((0, 0), (1, PAD_SEQ - 1 - N_CTX), (0, 0)))
        + jnp.pad(token_suffix,
                  ((0, 0), (1 + N_CTX, PAD_SEQ - SEQ_LEN), (0, 0)))
    )
    label3 = label.astype(jnp.int32).reshape(NW, 1, B // NW)
    out = _make_sc_kernel(B)(label3, table)
    return out[:, :SEQ_LEN, :]


# speedup vs baseline: 1.2861x; 1.0234x over previous
import functools

import jax
import jax.numpy as jnp
from jax import lax
from jax.experimental import pallas as pl
from jax.experimental.pallas import tpu as pltpu
from jax.experimental.pallas import tpu_sc as plsc

NUM_CLASSES = 1000
N_CTX = 16
CTX_DIM = 512
SEQ_LEN = 77
SUFFIX_LEN = SEQ_LEN - 1 - N_CTX
PAD_SEQ = 80

NC = 2
NS = 16
NW = NC * NS


def _make_sc_kernel(B: int):
    b_per_w = B // NW
    mesh = plsc.VectorSubcoreMesh(
        core_axis_name="c", subcore_axis_name="s", num_cores=NC, num_subcores=NS
    )

    @functools.partial(
        pl.kernel,
        out_type=jax.ShapeDtypeStruct((B, SEQ_LEN, CTX_DIM), jnp.float32),
        mesh=mesh,
        compiler_params=pltpu.CompilerParams(needs_layout_passes=False),
        scratch_types=[
            pltpu.VMEM((1, b_per_w), jnp.int32),
            pltpu.VMEM((16,), jnp.int32),
            pltpu.VMEM((16,), jnp.int32),
            pltpu.VMEM((1, PAD_SEQ, CTX_DIM), jnp.float32),
            pltpu.VMEM((1, PAD_SEQ, CTX_DIM), jnp.float32),
            pltpu.VMEM((1, 5, CTX_DIM), jnp.float32),
            pltpu.VMEM((1, 5, CTX_DIM), jnp.float32),
            pltpu.SemaphoreType.DMA,
            pltpu.SemaphoreType.DMA,
            pltpu.SemaphoreType.DMA,
            pltpu.SemaphoreType.DMA,
        ],
    )
    def body(label_hbm, table_hbm, out_hbm,
             idx_v, is0, is1, row0, row1, tail0, tail1,
             gsem0, gsem1, wsem0, wsem1):
        tail_v = (tail0, tail1)
        idx_s = (is0, is1)
        row_v = (row0, row1)
        gsem = (gsem0, gsem1)
        wsem = (wsem0, wsem1)
        wid = lax.axis_index("s") * NC + lax.axis_index("c")
        base = wid * b_per_w
        pltpu.sync_copy(label_hbm.at[wid], idx_v)
        zeros16 = jnp.zeros((16,), jnp.int32)

        def stage_idx(i, k):
            ivec = plsc.load_gather(
                idx_v, [zeros16, jnp.full((16,), i, jnp.int32)]
            )
            idx_s[k][...] = ivec

        def g_copy(k):
            return pltpu.make_async_copy(
                table_hbm.at[idx_s[k].at[pl.ds(0, 1)]], row_v[k], gsem[k]
            )

        def w_copy(i, k):
            return pltpu.make_async_copy(
                row_v[k].at[:, pl.ds(0, 72)],
                out_hbm.at[pl.ds(base + i, 1), pl.ds(0, 72)], wsem[k]
            )

        def t_copy(i, k):
            return pltpu.make_async_copy(
                tail_v[k],
                out_hbm.at[pl.ds(base + i, 1), pl.ds(72, 5)], wsem[k]
            )

        def fill_tail(k):
            for r in range(5):
                for t in range(CTX_DIM // 16):
                    tail_v[k][0, r, pl.ds(16 * t, 16)] = (
                        row_v[k][0, 72 + r, pl.ds(16 * t, 16)]
                    )

        for k in (0, 1):
            stage_idx(k, k)
            g_copy(k).start()

        def pair(g, _):
            for k in (0, 1):
                i = 2 * g + k
                g_copy(k).wait()
                wc = w_copy(i, k)
                wc.start()
                fill_tail(k)
                tc = t_copy(i, k)
                tc.start()
                wc.wait()
                tc.wait()

                @pl.when(i + 2 < b_per_w)
                def _():
                    stage_idx(i + 2, k)
                    g_copy(k).start()

            return 0

        lax.fori_loop(0, b_per_w // 2, pair, 0)

    return body


def kernel(label, cls_ctx, token_prefix, token_suffix):
    B = label.shape[0]
    table = (
        jnp.pad(jnp.broadcast_to(token_prefix, (NUM_CLASSES, 1, CTX_DIM)),
                ((0, 0), (0, PAD_SEQ - 1), (0, 0)))
        + jnp.pad(cls_ctx, ((0, 0), (1, PAD_SEQ - 1 - N_CTX), (0, 0)))
        + jnp.pad(token_suffix,
                  ((0, 0), (1 + N_CTX, PAD_SEQ - SEQ_LEN), (0, 0)))
    )
    label3 = label.astype(jnp.int32).reshape(NW, 1, B // NW)
    return _make_sc_kernel(B)(label3, table)
